# baseline (speedup 1.0000x reference)
import functools

import jax
import jax.numpy as jnp
from jax.experimental import pallas as pl
from jax.experimental.pallas import tpu as pltpu

VOCAB = 8192
ROWS = 8


def _body(xb_ref, yb_ref, *refs):
    row_refs = refs[:ROWS]
    out_ref, loss_ref = refs[ROWS], refs[ROWS + 1]
    i = pl.program_id(0)

    @pl.when(i == 0)
    def _():
        loss_ref[...] = jnp.zeros_like(loss_ref)

    col = jax.lax.broadcasted_iota(jnp.int32, (1, VOCAB), 1)
    acc = jnp.float32(0.0)
    for k in range(ROWS):
        row = row_refs[k][0]
        out_ref[k : k + 1, :] = row
        m = jnp.max(row)
        s = jnp.sum(jnp.exp(row - m))
        lse = m + jnp.log(s)
        y = yb_ref[i * ROWS + k]
        tgt = jnp.sum(jnp.where(col == y, row, 0.0))
        acc += lse - tgt
    loss_ref[...] += acc


def kernel(xb, yb, table):
    B, T = xb.shape
    N = B * T
    xf = xb.reshape(N).astype(jnp.int32)
    yf = yb.reshape(N).astype(jnp.int32)

    grid = (N // ROWS,)

    table3 = table.reshape(VOCAB, 1, VOCAB)

    def row_map(k):
        def index_map(i, xb_ref, yb_ref):
            return (xb_ref[i * ROWS + k], 0, 0)

        return index_map

    in_specs = [
        pl.BlockSpec((1, 1, VOCAB), row_map(k)) for k in range(ROWS)
    ]
    out_specs = [
        pl.BlockSpec((ROWS, VOCAB), lambda i, xb_ref, yb_ref: (i, 0)),
        pl.BlockSpec((1, 1), lambda i, xb_ref, yb_ref: (0, 0)),
    ]

    grid_spec = pltpu.PrefetchScalarGridSpec(
        num_scalar_prefetch=2,
        grid=grid,
        in_specs=in_specs,
        out_specs=out_specs,
    )

    out, loss_sum = pl.pallas_call(
        _body,
        grid_spec=grid_spec,
        out_shape=[
            jax.ShapeDtypeStruct((N, VOCAB), jnp.float32),
            jax.ShapeDtypeStruct((1, 1), jnp.float32),
        ],
        compiler_params=pltpu.CompilerParams(
            dimension_semantics=("arbitrary",),
        ),
    )(xf, yf, *([table3] * ROWS))

    logits = out.reshape(B, T, VOCAB)
    loss = loss_sum[0, 0] / N
    return (logits, loss)


# speedup vs baseline: 2.0055x; 2.0055x over previous
import jax
import jax.numpy as jnp
from jax.experimental import pallas as pl
from jax.experimental.pallas import tpu as pltpu

VOCAB = 8192
SUB, LANES = 8, VOCAB // 8
ROWS = 8


def _body(xb_ref, yb_ref, *refs):
    row_refs = refs[:ROWS]
    yv_ref = refs[ROWS]
    out_ref, loss_ref = refs[ROWS + 1], refs[ROWS + 2]
    i = pl.program_id(0)

    @pl.when(i == 0)
    def _():
        loss_ref[...] = jnp.zeros_like(loss_ref)

    for k in range(ROWS):
        out_ref[k] = row_refs[k][0]

    rows = out_ref[...]
    sub = jax.lax.broadcasted_iota(jnp.int32, (ROWS, SUB, LANES), 1)
    lane = jax.lax.broadcasted_iota(jnp.int32, (ROWS, SUB, LANES), 2)
    flat_idx = sub * LANES + lane

    m = jnp.max(rows, axis=(1, 2), keepdims=True)
    s = jnp.sum(jnp.exp(rows - m), axis=(1, 2), keepdims=True)
    lse = m + jnp.log(s)
    y = yv_ref[0, 0][:, None, None]
    tgt = jnp.sum(jnp.where(flat_idx == y, rows, 0.0), axis=(1, 2), keepdims=True)
    loss_ref[...] += jnp.sum(lse - tgt)


def kernel(xb, yb, table):
    B, T = xb.shape
    N = B * T
    xf = xb.reshape(N).astype(jnp.int32)
    yf = yb.reshape(N).astype(jnp.int32)

    table3 = table.reshape(VOCAB, SUB, LANES)

    grid = (N // ROWS,)

    def row_map(k):
        def index_map(i, xb_ref, yb_ref):
            return (xb_ref[i * ROWS + k], 0, 0)

        return index_map

    in_specs = [pl.BlockSpec((1, SUB, LANES), row_map(k)) for k in range(ROWS)]
    yv = yf.reshape(N // ROWS, 1, ROWS)
    in_specs.append(pl.BlockSpec((1, 1, ROWS), lambda i, xb_ref, yb_ref: (i, 0, 0)))
    out_specs = [
        pl.BlockSpec((ROWS, SUB, LANES), lambda i, xb_ref, yb_ref: (i, 0, 0)),
        pl.BlockSpec((1, 1), lambda i, xb_ref, yb_ref: (0, 0)),
    ]

    grid_spec = pltpu.PrefetchScalarGridSpec(
        num_scalar_prefetch=2,
        grid=grid,
        in_specs=in_specs,
        out_specs=out_specs,
    )

    out, loss_sum = pl.pallas_call(
        _body,
        grid_spec=grid_spec,
        out_shape=[
            jax.ShapeDtypeStruct((N, SUB, LANES), jnp.float32),
            jax.ShapeDtypeStruct((1, 1), jnp.float32),
        ],
        compiler_params=pltpu.CompilerParams(
            dimension_semantics=("arbitrary",),
        ),
    )(xf, yf, *([table3] * ROWS), yv)

    logits = out.reshape(B, T, VOCAB)
    loss = loss_sum[0, 0] / N
    return (logits, loss)


# speedup vs baseline: 2.6334x; 1.3131x over previous
import jax
import jax.numpy as jnp
from jax.experimental import pallas as pl
from jax.experimental.pallas import tpu as pltpu

VOCAB = 8192
SUB, LANES = 8, VOCAB // 8
ROWS = 16


def _body(xb_ref, yb_ref, *refs):
    row_refs = refs[:ROWS]
    yv_ref = refs[ROWS]
    out_ref, loss_ref = refs[ROWS + 1], refs[ROWS + 2]
    i = pl.program_id(0)

    @pl.when(i == 0)
    def _():
        loss_ref[...] = jnp.zeros_like(loss_ref)

    for k in range(ROWS):
        out_ref[k] = row_refs[k][0]

    rows = out_ref[...]
    sub = jax.lax.broadcasted_iota(jnp.int32, (ROWS, SUB, LANES), 1)
    lane = jax.lax.broadcasted_iota(jnp.int32, (ROWS, SUB, LANES), 2)
    flat_idx = sub * LANES + lane

    m = jnp.max(rows, axis=(1, 2), keepdims=True)
    s = jnp.sum(jnp.exp(rows - m), axis=(1, 2), keepdims=True)
    lse = m + jnp.log(s)
    y = yv_ref[0, 0][:, None, None]
    tgt = jnp.sum(jnp.where(flat_idx == y, rows, 0.0), axis=(1, 2), keepdims=True)
    loss_ref[...] += jnp.sum(lse - tgt)


def kernel(xb, yb, table):
    B, T = xb.shape
    N = B * T
    xf = xb.reshape(N).astype(jnp.int32)
    yf = yb.reshape(N).astype(jnp.int32)

    table3 = table.reshape(VOCAB, SUB, LANES)

    grid = (N // ROWS,)

    def row_map(k):
        def index_map(i, xb_ref, yb_ref):
            return (xb_ref[i * ROWS + k], 0, 0)

        return index_map

    in_specs = [pl.BlockSpec((1, SUB, LANES), row_map(k)) for k in range(ROWS)]
    yv = yf.reshape(N // ROWS, 1, ROWS)
    in_specs.append(pl.BlockSpec((1, 1, ROWS), lambda i, xb_ref, yb_ref: (i, 0, 0)))
    out_specs = [
        pl.BlockSpec((ROWS, SUB, LANES), lambda i, xb_ref, yb_ref: (i, 0, 0)),
        pl.BlockSpec((1, 1), lambda i, xb_ref, yb_ref: (0, 0)),
    ]

    grid_spec = pltpu.PrefetchScalarGridSpec(
        num_scalar_prefetch=2,
        grid=grid,
        in_specs=in_specs,
        out_specs=out_specs,
    )

    out, loss_sum = pl.pallas_call(
        _body,
        grid_spec=grid_spec,
        out_shape=[
            jax.ShapeDtypeStruct((N, SUB, LANES), jnp.float32),
            jax.ShapeDtypeStruct((1, 1), jnp.float32),
        ],
        compiler_params=pltpu.CompilerParams(
            dimension_semantics=("arbitrary",),
        ),
    )(xf, yf, *([table3] * ROWS), yv)

    logits = out.reshape(B, T, VOCAB)
    loss = loss_sum[0, 0] / N
    return (logits, loss)


# speedup vs baseline: 3.0023x; 1.1401x over previous
import jax
import jax.numpy as jnp
from jax.experimental import pallas as pl
from jax.experimental.pallas import tpu as pltpu

VOCAB = 8192
SUB, LANES = 8, VOCAB // 8
ROWS = 32


def _body(xb_ref, yb_ref, *refs):
    row_refs = refs[:ROWS]
    yv_ref = refs[ROWS]
    out_ref, loss_ref = refs[ROWS + 1], refs[ROWS + 2]
    i = pl.program_id(0)

    @pl.when(i == 0)
    def _():
        loss_ref[...] = jnp.zeros_like(loss_ref)

    for k in range(ROWS):
        out_ref[k] = row_refs[k][0]

    rows = out_ref[...]
    sub = jax.lax.broadcasted_iota(jnp.int32, (ROWS, SUB, LANES), 1)
    lane = jax.lax.broadcasted_iota(jnp.int32, (ROWS, SUB, LANES), 2)
    flat_idx = sub * LANES + lane

    m = jnp.max(rows, axis=(1, 2), keepdims=True)
    s = jnp.sum(jnp.exp(rows - m), axis=(1, 2), keepdims=True)
    lse = m + jnp.log(s)
    y = yv_ref[0, 0][:, None, None]
    tgt = jnp.sum(jnp.where(flat_idx == y, rows, 0.0), axis=(1, 2), keepdims=True)
    loss_ref[...] += jnp.sum(lse - tgt)


def kernel(xb, yb, table):
    B, T = xb.shape
    N = B * T
    xf = xb.reshape(N).astype(jnp.int32)
    yf = yb.reshape(N).astype(jnp.int32)

    table3 = table.reshape(VOCAB, SUB, LANES)

    grid = (N // ROWS,)

    def row_map(k):
        def index_map(i, xb_ref, yb_ref):
            return (xb_ref[i * ROWS + k], 0, 0)

        return index_map

    in_specs = [pl.BlockSpec((1, SUB, LANES), row_map(k)) for k in range(ROWS)]
    yv = yf.reshape(N // ROWS, 1, ROWS)
    in_specs.append(pl.BlockSpec((1, 1, ROWS), lambda i, xb_ref, yb_ref: (i, 0, 0)))
    out_specs = [
        pl.BlockSpec((ROWS, SUB, LANES), lambda i, xb_ref, yb_ref: (i, 0, 0)),
        pl.BlockSpec((1, 1), lambda i, xb_ref, yb_ref: (0, 0)),
    ]

    grid_spec = pltpu.PrefetchScalarGridSpec(
        num_scalar_prefetch=2,
        grid=grid,
        in_specs=in_specs,
        out_specs=out_specs,
    )

    out, loss_sum = pl.pallas_call(
        _body,
        grid_spec=grid_spec,
        out_shape=[
            jax.ShapeDtypeStruct((N, SUB, LANES), jnp.float32),
            jax.ShapeDtypeStruct((1, 1), jnp.float32),
        ],
        compiler_params=pltpu.CompilerParams(
            dimension_semantics=("arbitrary",),
        ),
    )(xf, yf, *([table3] * ROWS), yv)

    logits = out.reshape(B, T, VOCAB)
    loss = loss_sum[0, 0] / N
    return (logits, loss)


# speedup vs baseline: 3.1395x; 1.0457x over previous
import jax
import jax.numpy as jnp
from jax.experimental import pallas as pl
from jax.experimental.pallas import tpu as pltpu

VOCAB = 8192
SUB, LANES = 8, VOCAB // 8
ROWS = 64


def _body(xb_ref, yb_ref, *refs):
    row_refs = refs[:ROWS]
    yv_ref = refs[ROWS]
    out_ref, loss_ref = refs[ROWS + 1], refs[ROWS + 2]
    i = pl.program_id(0)

    @pl.when(i == 0)
    def _():
        loss_ref[...] = jnp.zeros_like(loss_ref)

    for k in range(ROWS):
        out_ref[k] = row_refs[k][0]

    rows = out_ref[...]
    sub = jax.lax.broadcasted_iota(jnp.int32, (ROWS, SUB, LANES), 1)
    lane = jax.lax.broadcasted_iota(jnp.int32, (ROWS, SUB, LANES), 2)
    flat_idx = sub * LANES + lane

    m = jnp.max(rows, axis=(1, 2), keepdims=True)
    s = jnp.sum(jnp.exp(rows - m), axis=(1, 2), keepdims=True)
    lse = m + jnp.log(s)
    y = yv_ref[0, 0][:, None, None]
    tgt = jnp.sum(jnp.where(flat_idx == y, rows, 0.0), axis=(1, 2), keepdims=True)
    loss_ref[...] += jnp.sum(lse - tgt)


def kernel(xb, yb, table):
    B, T = xb.shape
    N = B * T
    xf = xb.reshape(N).astype(jnp.int32)
    yf = yb.reshape(N).astype(jnp.int32)

    table3 = table.reshape(VOCAB, SUB, LANES)

    grid = (N // ROWS,)

    def row_map(k):
        def index_map(i, xb_ref, yb_ref):
            return (xb_ref[i * ROWS + k], 0, 0)

        return index_map

    in_specs = [pl.BlockSpec((1, SUB, LANES), row_map(k)) for k in range(ROWS)]
    yv = yf.reshape(N // ROWS, 1, ROWS)
    in_specs.append(pl.BlockSpec((1, 1, ROWS), lambda i, xb_ref, yb_ref: (i, 0, 0)))
    out_specs = [
        pl.BlockSpec((ROWS, SUB, LANES), lambda i, xb_ref, yb_ref: (i, 0, 0)),
        pl.BlockSpec((1, 1), lambda i, xb_ref, yb_ref: (0, 0)),
    ]

    grid_spec = pltpu.PrefetchScalarGridSpec(
        num_scalar_prefetch=2,
        grid=grid,
        in_specs=in_specs,
        out_specs=out_specs,
    )

    out, loss_sum = pl.pallas_call(
        _body,
        grid_spec=grid_spec,
        out_shape=[
            jax.ShapeDtypeStruct((N, SUB, LANES), jnp.float32),
            jax.ShapeDtypeStruct((1, 1), jnp.float32),
        ],
        compiler_params=pltpu.CompilerParams(
            dimension_semantics=("arbitrary",),
        ),
    )(xf, yf, *([table3] * ROWS), yv)

    logits = out.reshape(B, T, VOCAB)
    loss = loss_sum[0, 0] / N
    return (logits, loss)


# speedup vs baseline: 4.6015x; 1.4657x over previous
import jax
import jax.numpy as jnp
from jax import lax
from jax.experimental import pallas as pl
from jax.experimental.pallas import tpu as pltpu
from jax.experimental.pallas import tpu_sc as plsc

VOCAB = 8192
SUB, LANES = 8, VOCAB // 8
ROWS = 64

NC, NS = 2, 16
NW = NC * NS
CH = 2
NBUF = 4


def _sc_gather_body(xf2, table_hbm, out_hbm, idx_v, *rest):
    bufs = rest[:NBUF]
    gsem = rest[NBUF : 2 * NBUF]
    wsem = rest[2 * NBUF : 3 * NBUF]
    n = xf2.shape[0] * xf2.shape[1]
    bpw = n // NW
    nch = bpw // CH

    wid = lax.axis_index("s") * NC + lax.axis_index("c")
    pltpu.sync_copy(xf2.at[pl.ds(wid * nch, nch)], idx_v)

    def gather(k, b):
        return pltpu.make_async_copy(table_hbm.at[idx_v.at[k]], bufs[b], gsem[b])

    def wout(k, b):
        rows = (wid * nch + k) * CH
        return pltpu.make_async_copy(bufs[b], out_hbm.at[pl.ds(rows, CH)], wsem[b])

    for b in range(NBUF):
        gather(b, b).start()

    nrounds = nch // NBUF

    def round_(j, carry):
        for b in range(NBUF):
            k = j * NBUF + b
            gather(k, b).wait()
            wout(k, b).start()
            wout(k, b).wait()
            gather(k + NBUF, b).start()
        return carry

    lax.fori_loop(0, nrounds - 1, round_, 0)
    for b in range(NBUF):
        k = (nrounds - 1) * NBUF + b
        gather(k, b).wait()
        wout(k, b).start()
    for b in range(NBUF):
        wout((nrounds - 1) * NBUF + b, b).wait()


def _loss_body(xb_ref, yb_ref, *refs):
    row_refs = refs[:ROWS]
    yv_ref = refs[ROWS]
    loss_ref = refs[ROWS + 1]
    i = pl.program_id(0)

    @pl.when(i == 0)
    def _():
        loss_ref[...] = jnp.zeros_like(loss_ref)

    rows = jnp.concatenate([r[...] for r in row_refs], axis=0)
    sub = lax.broadcasted_iota(jnp.int32, (ROWS, SUB, LANES), 1)
    lane = lax.broadcasted_iota(jnp.int32, (ROWS, SUB, LANES), 2)
    flat_idx = sub * LANES + lane

    m = jnp.max(rows, axis=(1, 2), keepdims=True)
    s = jnp.sum(jnp.exp(rows - m), axis=(1, 2), keepdims=True)
    lse = m + jnp.log(s)
    y = yv_ref[0, 0][:, None, None]
    tgt = jnp.sum(jnp.where(flat_idx == y, rows, 0.0), axis=(1, 2), keepdims=True)
    loss_ref[...] += jnp.sum(lse - tgt)


def _tc_loss(xf, yf, table3, n):
    grid = (n // ROWS,)

    def row_map(k):
        def index_map(i, xb_ref, yb_ref):
            return (xb_ref[i * ROWS + k], 0, 0)

        return index_map

    in_specs = [pl.BlockSpec((1, SUB, LANES), row_map(k)) for k in range(ROWS)]
    yv = yf.reshape(n // ROWS, 1, ROWS)
    in_specs.append(pl.BlockSpec((1, 1, ROWS), lambda i, xb_ref, yb_ref: (i, 0, 0)))

    grid_spec = pltpu.PrefetchScalarGridSpec(
        num_scalar_prefetch=2,
        grid=grid,
        in_specs=in_specs,
        out_specs=[pl.BlockSpec((1, 1), lambda i, xb_ref, yb_ref: (0, 0))],
    )

    (loss_sum,) = pl.pallas_call(
        _loss_body,
        grid_spec=grid_spec,
        out_shape=[jax.ShapeDtypeStruct((1, 1), jnp.float32)],
        compiler_params=pltpu.CompilerParams(
            dimension_semantics=("arbitrary",),
        ),
    )(xf, yf, *([table3] * ROWS), yv)
    return loss_sum[0, 0] / n


def kernel(xb, yb, table):
    B, T = xb.shape
    N = B * T
    xf = xb.reshape(N).astype(jnp.int32)
    yf = yb.reshape(N).astype(jnp.int32)

    sc_gather = pl.kernel(
        _sc_gather_body,
        out_type=jax.ShapeDtypeStruct((N, VOCAB), jnp.float32),
        mesh=plsc.VectorSubcoreMesh(core_axis_name="c", subcore_axis_name="s"),
        scratch_types=(
            [pltpu.VMEM((N // NW // CH, CH), jnp.int32)]
            + [pltpu.VMEM((CH, VOCAB), jnp.float32)] * NBUF
            + [pltpu.SemaphoreType.DMA] * (2 * NBUF)
        ),
    )
    logits = sc_gather(xf.reshape(N // CH, CH), table)

    table3 = table.reshape(VOCAB, SUB, LANES)
    loss = _tc_loss(xf, yf, table3, N)
    return (logits.reshape(B, T, VOCAB), loss)
